# Initial kernel scaffold; baseline (speedup 1.0000x reference)
import jax
import jax.numpy as jnp
from jax.experimental import pallas as pl


def kernel(x, edge_index, id_index, W_self, W_id, W_neighbor, bias):
    raise NotImplementedError("write your pallas kernel here")


# speedup vs baseline: 10.9852x; 10.9852x over previous
import functools

import jax
import jax.numpy as jnp
from jax import lax
from jax.experimental import pallas as pl
from jax.experimental.pallas import tpu as pltpu
from jax.experimental.pallas import tpu_sc as plsc

NC = 2
NS = 16
NW = NC * NS

SLAB = 128
CHUNK_SLABS = 8


def _proj_body(x_ref, wn_ref, wi_ref, ws_ref, y_ref, z_ref, h_ref):
    xb = x_ref[...]
    y_ref[...] = jnp.dot(xb, wn_ref[...], preferred_element_type=jnp.float32)
    z_ref[...] = jnp.dot(xb, wi_ref[...], preferred_element_type=jnp.float32)
    h_ref[...] = jnp.dot(xb, ws_ref[...], preferred_element_type=jnp.float32)


def _combine_body(h_ref, nb_ref, cnt_ref, idp_ref, bias_ref, out_ref):
    ku = h_ref.shape[1]
    left = h_ref[...] + idp_ref[0] + idp_ref[1]
    cnt = cnt_ref[0, :, 0:1] + cnt_ref[1, :, 0:1]
    right = (nb_ref[0] + nb_ref[1]) / jnp.maximum(cnt, 1.0)
    bias = bias_ref[...]
    out_ref[:, 0:ku] = jax.nn.relu(left + bias[0, 0:ku])
    out_ref[:, ku:] = jax.nn.relu(right + bias[0, ku:])


def kernel(x, edge_index, id_index, W_self, W_id, W_neighbor, bias):
    n, d = x.shape
    ku = W_self.shape[1]
    e = edge_index.shape[1]
    nid = id_index.shape[0]

    n_slabs = e // SLAB
    n_chunks = n_slabs // CHUNK_SLABS
    tail_slabs = n_slabs - n_chunks * CHUNK_SLABS
    chunks_per_tile = -(-n_chunks // NW)
    npad = n + 8
    id_pad = -(-nid // (SLAB * CHUNK_SLABS)) * SLAB * CHUNK_SLABS
    id_chunks = id_pad // (SLAB * CHUNK_SLABS)
    rsub = 8 * ((n // NS) // 8)
    rlast = n - rsub * (NS - 1)

    blk = 2000
    grid1 = n // blk
    y, z, h = pl.pallas_call(
        _proj_body,
        grid=(grid1,),
        in_specs=[
            pl.BlockSpec((blk, d), lambda i: (i, 0)),
            pl.BlockSpec((d, ku), lambda i: (0, 0)),
            pl.BlockSpec((d, ku), lambda i: (0, 0)),
            pl.BlockSpec((d, ku), lambda i: (0, 0)),
        ],
        out_specs=[
            pl.BlockSpec((blk, ku), lambda i: (i, 0)),
            pl.BlockSpec((blk, ku), lambda i: (i, 0)),
            pl.BlockSpec((blk, ku), lambda i: (i, 0)),
        ],
        out_shape=[
            jax.ShapeDtypeStruct((n, ku), jnp.float32),
            jax.ShapeDtypeStruct((n, ku), jnp.float32),
            jax.ShapeDtypeStruct((n, ku), jnp.float32),
        ],
    )(x, W_neighbor, W_id, W_self)

    row2d = edge_index[0].astype(jnp.int32).reshape(n_slabs, SLAB)
    col2d = edge_index[1].astype(jnp.int32).reshape(n_slabs, SLAB)
    ids2d = jnp.concatenate(
        [id_index.astype(jnp.int32),
         jnp.full((id_pad - nid,), n, dtype=jnp.int32)]).reshape(
             id_chunks * CHUNK_SLABS, SLAB)
    zpad = jnp.concatenate([z, jnp.zeros((npad - n, ku), jnp.float32)])
    zeros64 = jnp.zeros((rlast, ku), jnp.float32)
    zeros16 = jnp.zeros((rlast, 16), jnp.float32)
    ones16 = jnp.ones((SLAB, 16), jnp.float32)

    mesh = plsc.VectorSubcoreMesh(
        core_axis_name="c", subcore_axis_name="s",
        num_cores=NC, num_subcores=NS)

    @functools.partial(
        pl.kernel,
        out_type=(
            jax.ShapeDtypeStruct((NC, n, ku), jnp.float32),
            jax.ShapeDtypeStruct((NC, n, 16), jnp.float32),
            jax.ShapeDtypeStruct((NC, n, ku), jnp.float32),
        ),
        mesh=mesh,
        compiler_params=pltpu.CompilerParams(use_tc_tiling_on_sc=False),
        scratch_types=[
            pltpu.VMEM((CHUNK_SLABS, SLAB), jnp.int32),
            pltpu.VMEM((CHUNK_SLABS, SLAB), jnp.int32),
            pltpu.VMEM((CHUNK_SLABS, SLAB, ku), jnp.float32),
            pltpu.VMEM((SLAB, 16), jnp.float32),
            pltpu.VMEM_SHARED((npad, ku), jnp.float32),
            pltpu.VMEM_SHARED((n, 16), jnp.float32),
            pltpu.SemaphoreType.DMA,
        ],
    )
    def sc_scatter(y_hbm, zp_hbm, row_hbm, col_hbm, ids_hbm, z64_hbm,
                   z16_hbm, ones_hbm, nb_out, cnt_out, idp_out,
                   row_v, col_v, rows_v, ones_v,
                   nb_acc, cnt_acc, sem):
        ci = lax.axis_index("c")
        si = lax.axis_index("s")
        wid = si * NC + ci

        r0 = si * rsub

        def zero_nb(nrows):
            pltpu.sync_copy(z64_hbm.at[pl.ds(0, nrows)],
                            nb_acc.at[pl.ds(r0, nrows)])

        def zero_accs(nrows):
            zero_nb(nrows)
            pltpu.sync_copy(z16_hbm.at[pl.ds(0, nrows)],
                            cnt_acc.at[pl.ds(r0, nrows)])

        @pl.when(si < NS - 1)
        def _():
            zero_accs(rsub)

        @pl.when(si == NS - 1)
        def _():
            zero_accs(rlast)

        pltpu.sync_copy(ones_hbm, ones_v)
        plsc.subcore_barrier()

        def do_slabs(nslabs):
            descs = []
            for j in range(nslabs):
                descs.append(pltpu.async_copy(
                    y_hbm.at[col_v.at[j]], rows_v.at[j], sem))
            for j in range(nslabs):
                descs[j].wait()
            for j in range(nslabs):
                pltpu.sync_copy(rows_v.at[j], nb_acc.at[row_v.at[j]],
                                add=True)
                pltpu.sync_copy(ones_v, cnt_acc.at[row_v.at[j]], add=True)

        @pl.loop(0, chunks_per_tile)
        def _(k):
            c = wid + k * NW

            @pl.when(c < n_chunks)
            def _():
                base = c * CHUNK_SLABS
                pltpu.sync_copy(row_hbm.at[pl.ds(base, CHUNK_SLABS)], row_v)
                pltpu.sync_copy(col_hbm.at[pl.ds(base, CHUNK_SLABS)], col_v)
                do_slabs(CHUNK_SLABS)

        @pl.when(wid == NW - 1)
        def _():
            base = n_chunks * CHUNK_SLABS
            pltpu.sync_copy(row_hbm.at[pl.ds(base, tail_slabs)],
                            row_v.at[pl.ds(0, tail_slabs)])
            pltpu.sync_copy(col_hbm.at[pl.ds(base, tail_slabs)],
                            col_v.at[pl.ds(0, tail_slabs)])
            do_slabs(tail_slabs)

        plsc.subcore_barrier()

        def writeout(dst, nrows):
            sl = pl.ds(r0, nrows)
            pltpu.sync_copy(nb_acc.at[sl], dst.at[ci].at[sl])

        @pl.when(si < NS - 1)
        def _():
            writeout(nb_out, rsub)
            pltpu.sync_copy(cnt_acc.at[pl.ds(r0, rsub)],
                            cnt_out.at[ci].at[pl.ds(r0, rsub)])
            zero_nb(rsub)

        @pl.when(si == NS - 1)
        def _():
            writeout(nb_out, rlast)
            pltpu.sync_copy(cnt_acc.at[pl.ds(r0, rlast)],
                            cnt_out.at[ci].at[pl.ds(r0, rlast)])
            zero_nb(rlast)

        plsc.subcore_barrier()

        @pl.when(wid < id_chunks)
        def _():
            base = wid * CHUNK_SLABS
            pltpu.sync_copy(ids_hbm.at[pl.ds(base, CHUNK_SLABS)], row_v)
            descs = []
            for j in range(CHUNK_SLABS):
                descs.append(pltpu.async_copy(
                    zp_hbm.at[row_v.at[j]], rows_v.at[j], sem))
            for j in range(CHUNK_SLABS):
                descs[j].wait()
            for j in range(CHUNK_SLABS):
                pltpu.sync_copy(rows_v.at[j], nb_acc.at[row_v.at[j]],
                                add=True)

        plsc.subcore_barrier()

        @pl.when(si < NS - 1)
        def _():
            writeout(idp_out, rsub)

        @pl.when(si == NS - 1)
        def _():
            writeout(idp_out, rlast)

    nb_p, cnt_p, idp_p = sc_scatter(y, zpad, row2d, col2d, ids2d,
                                    zeros64, zeros16, ones16)

    out = pl.pallas_call(
        _combine_body,
        grid=(grid1,),
        in_specs=[
            pl.BlockSpec((blk, ku), lambda i: (i, 0)),
            pl.BlockSpec((NC, blk, ku), lambda i: (0, i, 0)),
            pl.BlockSpec((NC, blk, 16), lambda i: (0, i, 0)),
            pl.BlockSpec((NC, blk, ku), lambda i: (0, i, 0)),
            pl.BlockSpec((1, 2 * ku), lambda i: (0, 0)),
        ],
        out_specs=pl.BlockSpec((blk, 2 * ku), lambda i: (i, 0)),
        out_shape=jax.ShapeDtypeStruct((n, 2 * ku), jnp.float32),
    )(h, nb_p, cnt_p, idp_p, bias.reshape(1, 2 * ku))
    return out
